# baseline (speedup 1.0000x reference)
import functools

import jax
import jax.numpy as jnp
from jax import lax
from jax.experimental import pallas as pl
from jax.experimental.pallas import tpu as pltpu
from jax.experimental.pallas import tpu_sc as plsc

_NC = 2
_NS = 16
_CHUNK = 128
_K = 5

_ROWB = 2000


def _sc_segment_sum(table, sd_idx, n_nodes):
    n_chunk_rows = sd_idx.shape[1]
    rows_per_tile = n_chunk_rows // _NS
    n_mac = rows_per_tile // _K
    out_rows = -(-n_nodes // (_NS * 8)) * 8
    spad = _NS * out_rows
    zrows = max(z for z in range(8, min(out_rows, 256) + 1, 8)
                if out_rows % z == 0)

    mesh = plsc.VectorSubcoreMesh(core_axis_name="c", subcore_axis_name="s",
                                  num_cores=_NC, num_subcores=_NS)

    zper = _K * _CHUNK

    @functools.partial(
        pl.kernel,
        out_type=jax.ShapeDtypeStruct((_NC, spad, 32), jnp.float32),
        mesh=mesh,
        compiler_params=pltpu.CompilerParams(use_tc_tiling_on_sc=False),
        scratch_types=[
            pltpu.VMEM((_K, 2, _CHUNK), jnp.int32),
            pltpu.VMEM((_K * _CHUNK, 32), jnp.float32),
            pltpu.SemaphoreType.DMA,
            pltpu.SemaphoreType.DMA,
            pltpu.VMEM_SHARED((spad, 32), jnp.float32),
        ],
    )
    def sc_pass(tab, sd_hbm, out, sd, rows, gsem, ssem, acc):
        c = lax.axis_index("c")
        s = lax.axis_index("s")

        zvec = jnp.zeros((16,), jnp.float32)

        def zrow(r, carry):
            rows[r, pl.ds(0, 16)] = zvec
            rows[r, pl.ds(16, 16)] = zvec
            return carry

        lax.fori_loop(0, zper, zrow, 0)
        zbase = s * out_rows
        nfull, rem = divmod(out_rows, zper)
        for k in range(nfull):
            pltpu.sync_copy(rows, acc.at[pl.ds(zbase + k * zper, zper), :])
        if rem:
            pltpu.sync_copy(rows.at[pl.ds(0, rem), :],
                            acc.at[pl.ds(zbase + nfull * zper, rem), :])
        plsc.subcore_barrier()

        tbase = s * rows_per_tile

        def mac(m, carry):
            off = tbase + m * _K
            pltpu.sync_copy(sd_hbm.at[c, pl.ds(off, _K), :, :], sd)
            gets = [
                pltpu.async_copy(tab.at[sd.at[j, 0]],
                                 rows.at[pl.ds(j * _CHUNK, _CHUNK), :], gsem)
                for j in range(_K)
            ]
            puts = []
            for j in range(_K):
                gets[j].wait()
                puts.append(
                    pltpu.async_copy(rows.at[pl.ds(j * _CHUNK, _CHUNK), :],
                                     acc.at[sd.at[j, 1]], ssem, add=True))
            for d in puts:
                d.wait()
            return carry

        lax.fori_loop(0, n_mac, mac, 0)
        plsc.subcore_barrier()

        obase = s * out_rows
        pltpu.sync_copy(acc.at[pl.ds(obase, out_rows), :],
                        out.at[c, pl.ds(obase, out_rows), :])

    return sc_pass(table, sd_idx)


def _sc_degree(dst_idx, n_nodes):
    n_chunk_rows = dst_idx.shape[0]
    rows_per_tile = n_chunk_rows // _NS
    n_mac = rows_per_tile // _K
    out_rows = -(-n_nodes // (_NS * 8)) * 8
    spad = _NS * out_rows
    zrows = max(z for z in range(8, min(out_rows, 256) + 1, 8)
                if out_rows % z == 0)

    mesh = plsc.VectorSubcoreMesh(core_axis_name="c", subcore_axis_name="s",
                                  num_cores=_NC, num_subcores=_NS)

    @functools.partial(
        pl.kernel,
        out_type=jax.ShapeDtypeStruct((_NC, spad, 32), jnp.float32),
        mesh=mesh,
        compiler_params=pltpu.CompilerParams(use_tc_tiling_on_sc=False),
        scratch_types=[
            pltpu.VMEM((_K, _CHUNK), jnp.int32),
            pltpu.VMEM((_CHUNK, 32), jnp.float32),
            pltpu.VMEM((zrows, 32), jnp.float32),
            pltpu.VMEM_SHARED((spad, 32), jnp.float32),
            pltpu.SemaphoreType.DMA,
        ],
    )
    def deg_pass(didx_hbm, out, didx, ones_v, zbuf, acc, ssem):
        c = lax.axis_index("c")
        s = lax.axis_index("s")

        zvec = jnp.zeros((16,), jnp.float32)
        ovec = jnp.ones((16,), jnp.float32)

        def zrow(r, carry):
            zbuf[r, pl.ds(0, 16)] = zvec
            zbuf[r, pl.ds(16, 16)] = zvec
            return carry

        def orow(r, carry):
            ones_v[r, pl.ds(0, 16)] = ovec
            ones_v[r, pl.ds(16, 16)] = ovec
            return carry

        lax.fori_loop(0, zrows, zrow, 0)
        lax.fori_loop(0, _CHUNK, orow, 0)
        zbase = s * out_rows
        for k in range(out_rows // zrows):
            pltpu.sync_copy(zbuf, acc.at[pl.ds(zbase + k * zrows, zrows), :])
        plsc.subcore_barrier()

        def mac(m, carry):
            off = s * rows_per_tile + m * _K
            pltpu.sync_copy(didx_hbm.at[pl.ds(off, _K), :], didx)
            puts = [
                pltpu.async_copy(ones_v, acc.at[didx.at[j]], ssem, add=True)
                for j in range(_K)
            ]
            for d in puts:
                d.wait()
            return carry

        lax.fori_loop(0, n_mac, mac, 0)
        plsc.subcore_barrier()

        obase = s * out_rows
        pltpu.sync_copy(acc.at[pl.ds(obase, out_rows), :],
                        out.at[c, pl.ds(obase, out_rows), :])

    return deg_pass(dst_idx)


def _gelu(x):
    return 0.5 * x * (1.0 + lax.erf(x * 0.7071067811865476))


def _tc_lin_stack(x, w, b):
    n = x.shape[0]
    grid = n // _ROWB

    def body(x_ref, w_ref, b_ref, out_ref):
        t = jnp.dot(x_ref[...], w_ref[...],
                    preferred_element_type=jnp.float32) + b_ref[...]
        out_ref[0] = t
        out_ref[1] = -t

    return pl.pallas_call(
        body,
        grid=(grid,),
        in_specs=[
            pl.BlockSpec((_ROWB, 64), lambda i: (i, 0)),
            pl.BlockSpec((64, 64), lambda i: (0, 0)),
            pl.BlockSpec((1, 64), lambda i: (0, 0)),
        ],
        out_specs=pl.BlockSpec((2, _ROWB, 64), lambda i: (0, i, 0)),
        out_shape=jax.ShapeDtypeStruct((2, n, 64), jnp.float32),
    )(x, w, b.reshape(1, 64))


def _tc_sage_h(sums2, taba, x, deg2):
    n = x.shape[0]
    grid = n // _ROWB

    def body(s2_ref, ta_ref, x_ref, d2_ref, h_ref, st_ref, sacc, qacc):
        i = pl.program_id(0)
        sums = jnp.concatenate([s2_ref[0], s2_ref[1]], axis=-1) + ta_ref[0]
        cnt = jnp.concatenate([d2_ref[0], d2_ref[1]], axis=-1) + 1.0
        h = sums / cnt + x_ref[...]
        h_ref[...] = h

        @pl.when(i == 0)
        def _():
            sacc[...] = jnp.zeros_like(sacc)
            qacc[...] = jnp.zeros_like(qacc)

        sacc[...] += jnp.sum(h, axis=0, keepdims=True)
        qacc[...] += jnp.sum(h * h, axis=0, keepdims=True)

        @pl.when(i == grid - 1)
        def _():
            st_ref[...] = jnp.concatenate([sacc[...], qacc[...]], axis=0)

    return pl.pallas_call(
        body,
        grid=(grid,),
        in_specs=[
            pl.BlockSpec((2, _ROWB, 32), lambda i: (0, i, 0)),
            pl.BlockSpec((1, _ROWB, 64), lambda i: (0, i, 0)),
            pl.BlockSpec((_ROWB, 64), lambda i: (i, 0)),
            pl.BlockSpec((2, _ROWB, 32), lambda i: (0, i, 0)),
        ],
        out_specs=[
            pl.BlockSpec((_ROWB, 64), lambda i: (i, 0)),
            pl.BlockSpec((2, 64), lambda i: (0, 0)),
        ],
        out_shape=[
            jax.ShapeDtypeStruct((n, 64), jnp.float32),
            jax.ShapeDtypeStruct((2, 64), jnp.float32),
        ],
        scratch_shapes=[
            pltpu.VMEM((1, 64), jnp.float32),
            pltpu.VMEM((1, 64), jnp.float32),
        ],
    )(sums2, taba, x, deg2)


def _tc_sage_norm(h, stats, gamma, beta, n_nodes):
    n = h.shape[0]
    grid = n // _ROWB
    inv_n = 1.0 / n_nodes

    def body(h_ref, st_ref, g_ref, b_ref, out_ref):
        mu = st_ref[pl.ds(0, 1), :] * inv_n
        var = st_ref[pl.ds(1, 1), :] * inv_n - mu * mu
        hn = (h_ref[...] - mu) * lax.rsqrt(var + 1e-5)
        out_ref[...] = _gelu(hn * g_ref[...] + b_ref[...])

    return pl.pallas_call(
        body,
        grid=(grid,),
        in_specs=[
            pl.BlockSpec((_ROWB, 64), lambda i: (i, 0)),
            pl.BlockSpec((2, 64), lambda i: (0, 0)),
            pl.BlockSpec((1, 64), lambda i: (0, 0)),
            pl.BlockSpec((1, 64), lambda i: (0, 0)),
        ],
        out_specs=pl.BlockSpec((_ROWB, 64), lambda i: (i, 0)),
        out_shape=jax.ShapeDtypeStruct((n, 64), jnp.float32),
    )(h, stats, gamma, beta)


def _tc_gin(x1, agg2, w1, b1, w2, b2, lw=None, lb=None):
    n = x1.shape[0]
    grid = n // _ROWB
    with_lin = lw is not None

    def body(x_ref, a2_ref, w1_ref, b1_ref, w2_ref, b2_ref, *rest):
        z = x_ref[...] + jnp.concatenate([a2_ref[0], a2_ref[1]], axis=-1)
        z1 = _gelu(jnp.dot(z, w1_ref[...],
                           preferred_element_type=jnp.float32) + b1_ref[...])
        x2 = jnp.dot(z1, w2_ref[...],
                     preferred_element_type=jnp.float32) + b2_ref[...]
        if with_lin:
            lw_ref, lb_ref, out_ref, tab_ref = rest
            t = jnp.dot(x2, lw_ref[...],
                        preferred_element_type=jnp.float32) + lb_ref[...]
            tab_ref[0] = t
            tab_ref[1] = -t
        else:
            (out_ref,) = rest
        out_ref[...] = x2

    in_specs = [
        pl.BlockSpec((_ROWB, 64), lambda i: (i, 0)),
        pl.BlockSpec((2, _ROWB, 32), lambda i: (0, i, 0)),
        pl.BlockSpec((64, 64), lambda i: (0, 0)),
        pl.BlockSpec((1, 64), lambda i: (0, 0)),
        pl.BlockSpec((64, 64), lambda i: (0, 0)),
        pl.BlockSpec((1, 64), lambda i: (0, 0)),
    ]
    args = [x1, agg2, w1, b1.reshape(1, 64), w2, b2.reshape(1, 64)]
    out_specs = [pl.BlockSpec((_ROWB, 64), lambda i: (i, 0))]
    out_shape = [jax.ShapeDtypeStruct((n, 64), jnp.float32)]
    if with_lin:
        in_specs += [pl.BlockSpec((64, 64), lambda i: (0, 0)),
                     pl.BlockSpec((1, 64), lambda i: (0, 0))]
        args += [lw, lb.reshape(1, 64)]
        out_specs += [pl.BlockSpec((2, _ROWB, 64), lambda i: (0, i, 0))]
        out_shape += [jax.ShapeDtypeStruct((2, n, 64), jnp.float32)]

    return pl.pallas_call(
        body,
        grid=(grid,),
        in_specs=in_specs,
        out_specs=out_specs,
        out_shape=out_shape,
    )(*args)


def _tc_fuse_readout(outs, fw, fb, ro):
    n = outs[0].shape[0]
    grid = n // _ROWB

    def ln(x, w, b):
        m = jnp.mean(x, axis=-1, keepdims=True)
        v = jnp.mean((x - m) * (x - m), axis=-1, keepdims=True)
        return (x - m) * lax.rsqrt(v + 1e-5) * w + b

    def body(o0, o1, o2, o3, o4, o5, fw_ref, fb_ref,
             w1_ref, b1_ref, l1w, l1b, w2_ref, b2_ref, l2w, l2b,
             w3_ref, b3_ref, xf_ref, p_ref):
        os_ = (o0, o1, o2, o3, o4, o5)
        xf = fb_ref[...] + jnp.zeros((o0.shape[0], 64), jnp.float32)
        for j in range(6):
            xf = xf + jnp.dot(os_[j][...], fw_ref[j],
                              preferred_element_type=jnp.float32)
        xf_ref[...] = xf
        h1 = jnp.maximum(
            ln(jnp.dot(xf, w1_ref[...], preferred_element_type=jnp.float32)
               + b1_ref[...], l1w[...], l1b[...]), 0.0)
        h2 = jnp.maximum(
            ln(jnp.dot(h1, w2_ref[...], preferred_element_type=jnp.float32)
               + b2_ref[...], l2w[...], l2b[...]), 0.0)
        logit = jnp.dot(h2, w3_ref[...],
                        preferred_element_type=jnp.float32) + b3_ref[...]
        p_ref[...] = 1.0 / (1.0 + jnp.exp(-logit))

    full = lambda shape: pl.BlockSpec(shape, lambda i: tuple(0 for _ in shape))
    rowspec = pl.BlockSpec((_ROWB, 64), lambda i: (i, 0))
    return pl.pallas_call(
        body,
        grid=(grid,),
        in_specs=[rowspec] * 6 + [
            full((6, 64, 64)), full((1, 64)),
            full((64, 128)), full((1, 128)), full((1, 128)), full((1, 128)),
            full((128, 128)), full((1, 128)), full((1, 128)), full((1, 128)),
            full((128, 1)), full((1, 1)),
        ],
        out_specs=[
            pl.BlockSpec((_ROWB, 64), lambda i: (i, 0)),
            pl.BlockSpec((_ROWB, 1), lambda i: (i, 0)),
        ],
        out_shape=[
            jax.ShapeDtypeStruct((n, 64), jnp.float32),
            jax.ShapeDtypeStruct((n, 1), jnp.float32),
        ],
    )(*outs, fw, fb.reshape(1, 64),
      ro['W1'], ro['b1'].reshape(1, 128), ro['ln1_w'].reshape(1, 128),
      ro['ln1_b'].reshape(1, 128),
      ro['W2'], ro['b2'].reshape(1, 128), ro['ln2_w'].reshape(1, 128),
      ro['ln2_b'].reshape(1, 128),
      ro['W3'], ro['b3'].reshape(1, 1))


def kernel(init_emb, edge_index_s, rate_b, params):
    n, d = init_emb.shape
    e = edge_index_s.shape[0]
    assert d == 64 and n % _ROWB == 0 and n % _NS == 0

    src = edge_index_s[:, 0].astype(jnp.int32)
    dst = edge_index_s[:, 1].astype(jnp.int32)
    sign = edge_index_s[:, 2].astype(jnp.int32)

    grain = _NS * _CHUNK * _K
    e_pad = ((e + grain - 1) // grain) * grain
    pad = e_pad - e
    src_p = jnp.concatenate([src, jnp.zeros((pad,), jnp.int32)])
    dst_p = jnp.concatenate([dst, jnp.full((pad,), n, jnp.int32)])
    sign_p = jnp.concatenate([sign, jnp.ones((pad,), jnp.int32)])
    neg = (sign_p < 0).astype(jnp.int32)

    n_chunk_rows = e_pad // _CHUNK
    cc = jnp.arange(_NC, dtype=jnp.int32).reshape(_NC, 1)
    src_sage = (((src_p + n * neg) * 2)[None, :] + cc).reshape(
        _NC, n_chunk_rows, _CHUNK)
    src_gin = ((src_p * 2)[None, :] + cc).reshape(_NC, n_chunk_rows, _CHUNK)
    dst_t = dst_p.reshape(n_chunk_rows, _CHUNK)
    dst2 = jnp.broadcast_to(dst_t[None], (_NC, n_chunk_rows, _CHUNK))
    sd_sage = jnp.stack([src_sage, dst2], axis=2)
    sd_gin = jnp.stack([src_gin, dst2], axis=2)

    deg2 = _sc_degree(dst_t, n)

    rb = rate_b.reshape(1, 1)
    x = init_emb
    outs = []
    taba = _tc_lin_stack(x, params['sage0']['lin_W'], params['sage0']['lin_b'])
    for i in range(3):
        sp = params['sage%d' % i]
        sums2 = _sc_segment_sum(taba.reshape(4 * n, 32), sd_sage, n)
        h, stats = _tc_sage_h(sums2, taba, x, deg2)
        gamma = (sp['norm_w'][None, :] + rb @ sp['rs_W'] + sp['rs_b'][None, :])
        beta = (sp['norm_b'][None, :] + rb @ sp['rb_W'] + sp['rb_b'][None, :])
        x1 = _tc_sage_norm(h, stats, gamma, beta, n)
        outs.append(x1)

        gp = params['gin%d' % i]
        agg2 = _sc_segment_sum(x1.reshape(n * 2, 32), sd_gin, n)
        if i < 2:
            nsp = params['sage%d' % (i + 1)]
            x2, taba = _tc_gin(x1, agg2, gp['W1'], gp['b1'], gp['W2'],
                               gp['b2'], nsp['lin_W'], nsp['lin_b'])
        else:
            (x2,) = _tc_gin(x1, agg2, gp['W1'], gp['b1'], gp['W2'], gp['b2'])
        outs.append(x2)
        x = x2

    fw = params['fuse_W'].reshape(6, 64, 64)
    x_final, prob = _tc_fuse_readout(outs, fw, params['fuse_b'], params['ro'])
    return x_final, prob


# speedup vs baseline: 1.1811x; 1.1811x over previous
import functools

import jax
import jax.numpy as jnp
from jax import lax
from jax.experimental import pallas as pl
from jax.experimental.pallas import tpu as pltpu
from jax.experimental.pallas import tpu_sc as plsc

_NC = 2
_NS = 16
_CHUNK = 128
_K = 4

_ROWB = 2000


def _sc_segment_sum(table, sd_idx, n_nodes):
    n_chunk_rows = sd_idx.shape[1]
    rows_per_tile = n_chunk_rows // _NS
    n_mac = rows_per_tile // _K
    out_rows = -(-n_nodes // (_NS * 8)) * 8
    spad = _NS * out_rows
    zrows = max(z for z in range(8, min(out_rows, 256) + 1, 8)
                if out_rows % z == 0)

    mesh = plsc.VectorSubcoreMesh(core_axis_name="c", subcore_axis_name="s",
                                  num_cores=_NC, num_subcores=_NS)

    zper = _K * _CHUNK

    @functools.partial(
        pl.kernel,
        out_type=jax.ShapeDtypeStruct((_NC, spad, 32), jnp.float32),
        mesh=mesh,
        compiler_params=pltpu.CompilerParams(use_tc_tiling_on_sc=False),
        scratch_types=[
            pltpu.VMEM((_K, 2, _CHUNK), jnp.int32),
            pltpu.VMEM((_K * _CHUNK, 32), jnp.float32),
            pltpu.SemaphoreType.DMA,
            pltpu.SemaphoreType.DMA,
            pltpu.VMEM_SHARED((spad, 32), jnp.float32),
        ],
    )
    def sc_pass(tab, sd_hbm, out, sd, rows, gsem, ssem, acc):
        c = lax.axis_index("c")
        s = lax.axis_index("s")

        zvec = jnp.zeros((16,), jnp.float32)

        def zrow(r, carry):
            rows[r, pl.ds(0, 16)] = zvec
            rows[r, pl.ds(16, 16)] = zvec
            return carry

        lax.fori_loop(0, zper, zrow, 0)
        zbase = s * out_rows
        nfull, rem = divmod(out_rows, zper)
        for k in range(nfull):
            pltpu.sync_copy(rows, acc.at[pl.ds(zbase + k * zper, zper), :])
        if rem:
            pltpu.sync_copy(rows.at[pl.ds(0, rem), :],
                            acc.at[pl.ds(zbase + nfull * zper, rem), :])
        plsc.subcore_barrier()

        tbase = s * rows_per_tile

        def mac(m, carry):
            off = tbase + m * _K
            pltpu.sync_copy(sd_hbm.at[c, pl.ds(off, _K), :, :], sd)
            gets = [
                pltpu.async_copy(tab.at[sd.at[j, 0]],
                                 rows.at[pl.ds(j * _CHUNK, _CHUNK), :], gsem)
                for j in range(_K)
            ]
            puts = []
            for j in range(_K):
                gets[j].wait()
                puts.append(
                    pltpu.async_copy(rows.at[pl.ds(j * _CHUNK, _CHUNK), :],
                                     acc.at[sd.at[j, 1]], ssem, add=True))
            for d in puts:
                d.wait()
            return carry

        lax.fori_loop(0, n_mac, mac, 0)
        plsc.subcore_barrier()

        obase = s * out_rows
        pltpu.sync_copy(acc.at[pl.ds(obase, out_rows), :],
                        out.at[c, pl.ds(obase, out_rows), :])

    return sc_pass(table, sd_idx)


def _sc_degree(dst_idx, n_nodes):
    n_chunk_rows = dst_idx.shape[0]
    rows_per_tile = n_chunk_rows // _NS
    n_mac = rows_per_tile // _K
    out_rows = -(-n_nodes // (_NS * 8)) * 8
    spad = _NS * out_rows
    zrows = max(z for z in range(8, min(out_rows, 256) + 1, 8)
                if out_rows % z == 0)

    mesh = plsc.VectorSubcoreMesh(core_axis_name="c", subcore_axis_name="s",
                                  num_cores=_NC, num_subcores=_NS)

    @functools.partial(
        pl.kernel,
        out_type=jax.ShapeDtypeStruct((_NC, spad, 32), jnp.float32),
        mesh=mesh,
        compiler_params=pltpu.CompilerParams(use_tc_tiling_on_sc=False),
        scratch_types=[
            pltpu.VMEM((_K, _CHUNK), jnp.int32),
            pltpu.VMEM((_CHUNK, 32), jnp.float32),
            pltpu.VMEM((zrows, 32), jnp.float32),
            pltpu.VMEM_SHARED((spad, 32), jnp.float32),
            pltpu.SemaphoreType.DMA,
        ],
    )
    def deg_pass(didx_hbm, out, didx, ones_v, zbuf, acc, ssem):
        c = lax.axis_index("c")
        s = lax.axis_index("s")

        zvec = jnp.zeros((16,), jnp.float32)
        ovec = jnp.ones((16,), jnp.float32)

        def zrow(r, carry):
            zbuf[r, pl.ds(0, 16)] = zvec
            zbuf[r, pl.ds(16, 16)] = zvec
            return carry

        def orow(r, carry):
            ones_v[r, pl.ds(0, 16)] = ovec
            ones_v[r, pl.ds(16, 16)] = ovec
            return carry

        lax.fori_loop(0, zrows, zrow, 0)
        lax.fori_loop(0, _CHUNK, orow, 0)
        zbase = s * out_rows
        for k in range(out_rows // zrows):
            pltpu.sync_copy(zbuf, acc.at[pl.ds(zbase + k * zrows, zrows), :])
        plsc.subcore_barrier()

        def mac(m, carry):
            off = s * rows_per_tile + m * _K
            pltpu.sync_copy(didx_hbm.at[pl.ds(off, _K), :], didx)
            puts = [
                pltpu.async_copy(ones_v, acc.at[didx.at[j]], ssem, add=True)
                for j in range(_K)
            ]
            for d in puts:
                d.wait()
            return carry

        lax.fori_loop(0, n_mac, mac, 0)
        plsc.subcore_barrier()

        obase = s * out_rows
        pltpu.sync_copy(acc.at[pl.ds(obase, out_rows), :],
                        out.at[c, pl.ds(obase, out_rows), :])

    return deg_pass(dst_idx)


def _gelu(x):
    return 0.5 * x * (1.0 + lax.erf(x * 0.7071067811865476))


def _tc_lin_stack(x, w, b):
    n = x.shape[0]
    grid = n // _ROWB

    def body(x_ref, w_ref, b_ref, out_ref):
        t = jnp.dot(x_ref[...], w_ref[...],
                    preferred_element_type=jnp.float32) + b_ref[...]
        out_ref[0] = t
        out_ref[1] = -t

    return pl.pallas_call(
        body,
        grid=(grid,),
        in_specs=[
            pl.BlockSpec((_ROWB, 64), lambda i: (i, 0)),
            pl.BlockSpec((64, 64), lambda i: (0, 0)),
            pl.BlockSpec((1, 64), lambda i: (0, 0)),
        ],
        out_specs=pl.BlockSpec((2, _ROWB, 64), lambda i: (0, i, 0)),
        out_shape=jax.ShapeDtypeStruct((2, n, 64), jnp.float32),
    )(x, w, b.reshape(1, 64))


def _tc_sage_h(sums2, taba, x, deg2):
    n = x.shape[0]
    grid = n // _ROWB

    def body(s2_ref, ta_ref, x_ref, d2_ref, h_ref, st_ref, sacc, qacc):
        i = pl.program_id(0)
        sums = jnp.concatenate([s2_ref[0], s2_ref[1]], axis=-1) + ta_ref[0]
        cnt = jnp.concatenate([d2_ref[0], d2_ref[1]], axis=-1) + 1.0
        h = sums / cnt + x_ref[...]
        h_ref[...] = h

        @pl.when(i == 0)
        def _():
            sacc[...] = jnp.zeros_like(sacc)
            qacc[...] = jnp.zeros_like(qacc)

        sacc[...] += jnp.sum(h, axis=0, keepdims=True)
        qacc[...] += jnp.sum(h * h, axis=0, keepdims=True)

        @pl.when(i == grid - 1)
        def _():
            st_ref[...] = jnp.concatenate([sacc[...], qacc[...]], axis=0)

    return pl.pallas_call(
        body,
        grid=(grid,),
        in_specs=[
            pl.BlockSpec((2, _ROWB, 32), lambda i: (0, i, 0)),
            pl.BlockSpec((1, _ROWB, 64), lambda i: (0, i, 0)),
            pl.BlockSpec((_ROWB, 64), lambda i: (i, 0)),
            pl.BlockSpec((2, _ROWB, 32), lambda i: (0, i, 0)),
        ],
        out_specs=[
            pl.BlockSpec((_ROWB, 64), lambda i: (i, 0)),
            pl.BlockSpec((2, 64), lambda i: (0, 0)),
        ],
        out_shape=[
            jax.ShapeDtypeStruct((n, 64), jnp.float32),
            jax.ShapeDtypeStruct((2, 64), jnp.float32),
        ],
        scratch_shapes=[
            pltpu.VMEM((1, 64), jnp.float32),
            pltpu.VMEM((1, 64), jnp.float32),
        ],
    )(sums2, taba, x, deg2)


def _tc_sage_norm(h, stats, gamma, beta, n_nodes):
    n = h.shape[0]
    grid = n // _ROWB
    inv_n = 1.0 / n_nodes

    def body(h_ref, st_ref, g_ref, b_ref, out_ref):
        mu = st_ref[pl.ds(0, 1), :] * inv_n
        var = st_ref[pl.ds(1, 1), :] * inv_n - mu * mu
        hn = (h_ref[...] - mu) * lax.rsqrt(var + 1e-5)
        out_ref[...] = _gelu(hn * g_ref[...] + b_ref[...])

    return pl.pallas_call(
        body,
        grid=(grid,),
        in_specs=[
            pl.BlockSpec((_ROWB, 64), lambda i: (i, 0)),
            pl.BlockSpec((2, 64), lambda i: (0, 0)),
            pl.BlockSpec((1, 64), lambda i: (0, 0)),
            pl.BlockSpec((1, 64), lambda i: (0, 0)),
        ],
        out_specs=pl.BlockSpec((_ROWB, 64), lambda i: (i, 0)),
        out_shape=jax.ShapeDtypeStruct((n, 64), jnp.float32),
    )(h, stats, gamma, beta)


def _tc_gin(x1, agg2, w1, b1, w2, b2, lw=None, lb=None):
    n = x1.shape[0]
    grid = n // _ROWB
    with_lin = lw is not None

    def body(x_ref, a2_ref, w1_ref, b1_ref, w2_ref, b2_ref, *rest):
        z = x_ref[...] + jnp.concatenate([a2_ref[0], a2_ref[1]], axis=-1)
        z1 = _gelu(jnp.dot(z, w1_ref[...],
                           preferred_element_type=jnp.float32) + b1_ref[...])
        x2 = jnp.dot(z1, w2_ref[...],
                     preferred_element_type=jnp.float32) + b2_ref[...]
        if with_lin:
            lw_ref, lb_ref, out_ref, tab_ref = rest
            t = jnp.dot(x2, lw_ref[...],
                        preferred_element_type=jnp.float32) + lb_ref[...]
            tab_ref[0] = t
            tab_ref[1] = -t
        else:
            (out_ref,) = rest
        out_ref[...] = x2

    in_specs = [
        pl.BlockSpec((_ROWB, 64), lambda i: (i, 0)),
        pl.BlockSpec((2, _ROWB, 32), lambda i: (0, i, 0)),
        pl.BlockSpec((64, 64), lambda i: (0, 0)),
        pl.BlockSpec((1, 64), lambda i: (0, 0)),
        pl.BlockSpec((64, 64), lambda i: (0, 0)),
        pl.BlockSpec((1, 64), lambda i: (0, 0)),
    ]
    args = [x1, agg2, w1, b1.reshape(1, 64), w2, b2.reshape(1, 64)]
    out_specs = [pl.BlockSpec((_ROWB, 64), lambda i: (i, 0))]
    out_shape = [jax.ShapeDtypeStruct((n, 64), jnp.float32)]
    if with_lin:
        in_specs += [pl.BlockSpec((64, 64), lambda i: (0, 0)),
                     pl.BlockSpec((1, 64), lambda i: (0, 0))]
        args += [lw, lb.reshape(1, 64)]
        out_specs += [pl.BlockSpec((2, _ROWB, 64), lambda i: (0, i, 0))]
        out_shape += [jax.ShapeDtypeStruct((2, n, 64), jnp.float32)]

    return pl.pallas_call(
        body,
        grid=(grid,),
        in_specs=in_specs,
        out_specs=out_specs,
        out_shape=out_shape,
    )(*args)


def _tc_fuse_readout(outs, fw, fb, ro):
    n = outs[0].shape[0]
    grid = n // _ROWB

    def ln(x, w, b):
        m = jnp.mean(x, axis=-1, keepdims=True)
        v = jnp.mean((x - m) * (x - m), axis=-1, keepdims=True)
        return (x - m) * lax.rsqrt(v + 1e-5) * w + b

    def body(o0, o1, o2, o3, o4, o5, fw_ref, fb_ref,
             w1_ref, b1_ref, l1w, l1b, w2_ref, b2_ref, l2w, l2b,
             w3_ref, b3_ref, xf_ref, p_ref):
        os_ = (o0, o1, o2, o3, o4, o5)
        xf = fb_ref[...] + jnp.zeros((o0.shape[0], 64), jnp.float32)
        for j in range(6):
            xf = xf + jnp.dot(os_[j][...], fw_ref[j],
                              preferred_element_type=jnp.float32)
        xf_ref[...] = xf
        h1 = jnp.maximum(
            ln(jnp.dot(xf, w1_ref[...], preferred_element_type=jnp.float32)
               + b1_ref[...], l1w[...], l1b[...]), 0.0)
        h2 = jnp.maximum(
            ln(jnp.dot(h1, w2_ref[...], preferred_element_type=jnp.float32)
               + b2_ref[...], l2w[...], l2b[...]), 0.0)
        logit = jnp.dot(h2, w3_ref[...],
                        preferred_element_type=jnp.float32) + b3_ref[...]
        p_ref[...] = 1.0 / (1.0 + jnp.exp(-logit))

    full = lambda shape: pl.BlockSpec(shape, lambda i: tuple(0 for _ in shape))
    rowspec = pl.BlockSpec((_ROWB, 64), lambda i: (i, 0))
    return pl.pallas_call(
        body,
        grid=(grid,),
        in_specs=[rowspec] * 6 + [
            full((6, 64, 64)), full((1, 64)),
            full((64, 128)), full((1, 128)), full((1, 128)), full((1, 128)),
            full((128, 128)), full((1, 128)), full((1, 128)), full((1, 128)),
            full((128, 1)), full((1, 1)),
        ],
        out_specs=[
            pl.BlockSpec((_ROWB, 64), lambda i: (i, 0)),
            pl.BlockSpec((_ROWB, 1), lambda i: (i, 0)),
        ],
        out_shape=[
            jax.ShapeDtypeStruct((n, 64), jnp.float32),
            jax.ShapeDtypeStruct((n, 1), jnp.float32),
        ],
    )(*outs, fw, fb.reshape(1, 64),
      ro['W1'], ro['b1'].reshape(1, 128), ro['ln1_w'].reshape(1, 128),
      ro['ln1_b'].reshape(1, 128),
      ro['W2'], ro['b2'].reshape(1, 128), ro['ln2_w'].reshape(1, 128),
      ro['ln2_b'].reshape(1, 128),
      ro['W3'], ro['b3'].reshape(1, 1))


def kernel(init_emb, edge_index_s, rate_b, params):
    n, d = init_emb.shape
    e = edge_index_s.shape[0]
    assert d == 64 and n % _ROWB == 0 and n % _NS == 0

    src = edge_index_s[:, 0].astype(jnp.int32)
    dst = edge_index_s[:, 1].astype(jnp.int32)
    sign = edge_index_s[:, 2].astype(jnp.int32)

    grain = _NS * _CHUNK * _K
    e_pad = ((e + grain - 1) // grain) * grain
    pad = e_pad - e
    src_p = jnp.concatenate([src, jnp.zeros((pad,), jnp.int32)])
    dst_p = jnp.concatenate([dst, jnp.full((pad,), n, jnp.int32)])
    sign_p = jnp.concatenate([sign, jnp.ones((pad,), jnp.int32)])
    neg = (sign_p < 0).astype(jnp.int32)

    n_chunk_rows = e_pad // _CHUNK
    cc = jnp.arange(_NC, dtype=jnp.int32).reshape(_NC, 1)
    src_sage = (((src_p + n * neg) * 2)[None, :] + cc).reshape(
        _NC, n_chunk_rows, _CHUNK)
    src_gin = ((src_p * 2)[None, :] + cc).reshape(_NC, n_chunk_rows, _CHUNK)
    dst_t = dst_p.reshape(n_chunk_rows, _CHUNK)
    dst2 = jnp.broadcast_to(dst_t[None], (_NC, n_chunk_rows, _CHUNK))
    sd_sage = jnp.stack([src_sage, dst2], axis=2)
    sd_gin = jnp.stack([src_gin, dst2], axis=2)

    deg2 = _sc_degree(dst_t, n)

    rb = rate_b.reshape(1, 1)
    x = init_emb
    outs = []
    taba = _tc_lin_stack(x, params['sage0']['lin_W'], params['sage0']['lin_b'])
    for i in range(3):
        sp = params['sage%d' % i]
        sums2 = _sc_segment_sum(taba.reshape(4 * n, 32), sd_sage, n)
        h, stats = _tc_sage_h(sums2, taba, x, deg2)
        gamma = (sp['norm_w'][None, :] + rb @ sp['rs_W'] + sp['rs_b'][None, :])
        beta = (sp['norm_b'][None, :] + rb @ sp['rb_W'] + sp['rb_b'][None, :])
        x1 = _tc_sage_norm(h, stats, gamma, beta, n)
        outs.append(x1)

        gp = params['gin%d' % i]
        agg2 = _sc_segment_sum(x1.reshape(n * 2, 32), sd_gin, n)
        if i < 2:
            nsp = params['sage%d' % (i + 1)]
            x2, taba = _tc_gin(x1, agg2, gp['W1'], gp['b1'], gp['W2'],
                               gp['b2'], nsp['lin_W'], nsp['lin_b'])
        else:
            (x2,) = _tc_gin(x1, agg2, gp['W1'], gp['b1'], gp['W2'], gp['b2'])
        outs.append(x2)
        x = x2

    fw = params['fuse_W'].reshape(6, 64, 64)
    x_final, prob = _tc_fuse_readout(outs, fw, params['fuse_b'], params['ro'])
    return x_final, prob


# speedup vs baseline: 1.2015x; 1.0173x over previous
import functools

import jax
import jax.numpy as jnp
from jax import lax
from jax.experimental import pallas as pl
from jax.experimental.pallas import tpu as pltpu
from jax.experimental.pallas import tpu_sc as plsc

_NC = 2
_NS = 16
_CHUNK = 128
_K = 4

_ROWB = 2000


def _sc_segment_sum(table, sd_idx, n_nodes):
    n_chunk_rows = sd_idx.shape[1]
    rows_per_tile = n_chunk_rows // _NS
    n_mac = rows_per_tile // _K
    out_rows = -(-n_nodes // (_NS * 8)) * 8
    spad = _NS * out_rows
    zrows = max(z for z in range(8, min(out_rows, 256) + 1, 8)
                if out_rows % z == 0)

    mesh = plsc.VectorSubcoreMesh(core_axis_name="c", subcore_axis_name="s",
                                  num_cores=_NC, num_subcores=_NS)

    zper = _K * _CHUNK

    @functools.partial(
        pl.kernel,
        out_type=jax.ShapeDtypeStruct((_NC, spad, 32), jnp.float32),
        mesh=mesh,
        compiler_params=pltpu.CompilerParams(use_tc_tiling_on_sc=False),
        scratch_types=[
            pltpu.VMEM((_K, 2, _CHUNK), jnp.int32),
            pltpu.VMEM((_K * _CHUNK, 32), jnp.float32),
            pltpu.SemaphoreType.DMA,
            pltpu.SemaphoreType.DMA,
            pltpu.VMEM_SHARED((spad, 32), jnp.float32),
        ],
    )
    def sc_pass(tab, sd_hbm, out, sd, rows, gsem, ssem, acc):
        c = lax.axis_index("c")
        s = lax.axis_index("s")

        zvec = jnp.zeros((16,), jnp.float32)

        def zrow(r, carry):
            rows[r, pl.ds(0, 16)] = zvec
            rows[r, pl.ds(16, 16)] = zvec
            return carry

        lax.fori_loop(0, zper, zrow, 0)
        zbase = s * out_rows
        nfull, rem = divmod(out_rows, zper)
        for k in range(nfull):
            pltpu.sync_copy(rows, acc.at[pl.ds(zbase + k * zper, zper), :])
        if rem:
            pltpu.sync_copy(rows.at[pl.ds(0, rem), :],
                            acc.at[pl.ds(zbase + nfull * zper, rem), :])
        plsc.subcore_barrier()

        tbase = s * rows_per_tile

        def mac(m, carry):
            off = tbase + m * _K
            pltpu.sync_copy(sd_hbm.at[c, pl.ds(off, _K), :, :], sd)
            gets = [
                pltpu.async_copy(tab.at[sd.at[j, 0]],
                                 rows.at[pl.ds(j * _CHUNK, _CHUNK), :], gsem)
                for j in range(_K)
            ]
            puts = []
            for j in range(_K):
                gets[j].wait()
                puts.append(
                    pltpu.async_copy(rows.at[pl.ds(j * _CHUNK, _CHUNK), :],
                                     acc.at[sd.at[j, 1]], ssem, add=True))
            for d in puts:
                d.wait()
            return carry

        lax.fori_loop(0, n_mac, mac, 0)
        plsc.subcore_barrier()

        obase = s * out_rows
        pltpu.sync_copy(acc.at[pl.ds(obase, out_rows), :],
                        out.at[c, pl.ds(obase, out_rows), :])

    return sc_pass(table, sd_idx)


def _sc_degree(dst_idx, n_nodes):
    n_chunk_rows = dst_idx.shape[0]
    rows_per_tile = n_chunk_rows // _NS
    n_mac = rows_per_tile // _K
    out_rows = -(-n_nodes // (_NS * 8)) * 8
    spad = _NS * out_rows
    zrows = max(z for z in range(8, min(out_rows, 256) + 1, 8)
                if out_rows % z == 0)

    mesh = plsc.VectorSubcoreMesh(core_axis_name="c", subcore_axis_name="s",
                                  num_cores=_NC, num_subcores=_NS)

    @functools.partial(
        pl.kernel,
        out_type=jax.ShapeDtypeStruct((_NC, spad, 32), jnp.float32),
        mesh=mesh,
        compiler_params=pltpu.CompilerParams(use_tc_tiling_on_sc=False),
        scratch_types=[
            pltpu.VMEM((_K, _CHUNK), jnp.int32),
            pltpu.VMEM((_CHUNK, 32), jnp.float32),
            pltpu.VMEM((zrows, 32), jnp.float32),
            pltpu.VMEM_SHARED((spad, 32), jnp.float32),
            pltpu.SemaphoreType.DMA,
        ],
    )
    def deg_pass(didx_hbm, out, didx, ones_v, zbuf, acc, ssem):
        c = lax.axis_index("c")
        s = lax.axis_index("s")

        zvec = jnp.zeros((16,), jnp.float32)
        ovec = jnp.ones((16,), jnp.float32)

        def zrow(r, carry):
            zbuf[r, pl.ds(0, 16)] = zvec
            zbuf[r, pl.ds(16, 16)] = zvec
            return carry

        def orow(r, carry):
            ones_v[r, pl.ds(0, 16)] = ovec
            ones_v[r, pl.ds(16, 16)] = ovec
            return carry

        lax.fori_loop(0, zrows, zrow, 0)
        lax.fori_loop(0, _CHUNK, orow, 0)
        zbase = s * out_rows
        for k in range(out_rows // zrows):
            pltpu.sync_copy(zbuf, acc.at[pl.ds(zbase + k * zrows, zrows), :])
        plsc.subcore_barrier()

        def mac(m, carry):
            off = s * rows_per_tile + m * _K
            pltpu.sync_copy(didx_hbm.at[pl.ds(off, _K), :], didx)
            puts = [
                pltpu.async_copy(ones_v, acc.at[didx.at[j]], ssem, add=True)
                for j in range(_K)
            ]
            for d in puts:
                d.wait()
            return carry

        lax.fori_loop(0, n_mac, mac, 0)
        plsc.subcore_barrier()

        obase = s * out_rows
        pltpu.sync_copy(acc.at[pl.ds(obase, out_rows), :],
                        out.at[c, pl.ds(obase, out_rows), :])

    return deg_pass(dst_idx)


def _gelu(x):
    return 0.5 * x * (1.0 + lax.erf(x * 0.7071067811865476))


def _tc_lin_stack(x, w, b):
    n = x.shape[0]
    grid = n // _ROWB

    def body(x_ref, w_ref, b_ref, out_ref):
        t = jnp.dot(x_ref[...], w_ref[...],
                    preferred_element_type=jnp.float32) + b_ref[...]
        out_ref[0] = t
        out_ref[1] = -t

    return pl.pallas_call(
        body,
        grid=(grid,),
        in_specs=[
            pl.BlockSpec((_ROWB, 64), lambda i: (i, 0)),
            pl.BlockSpec((64, 64), lambda i: (0, 0)),
            pl.BlockSpec((1, 64), lambda i: (0, 0)),
        ],
        out_specs=pl.BlockSpec((2, _ROWB, 64), lambda i: (0, i, 0)),
        out_shape=jax.ShapeDtypeStruct((2, n, 64), jnp.float32),
    )(x, w, b.reshape(1, 64))


def _tc_sage_dense(sums2, x, deg2, lw, lb, gamma, beta, n_nodes):
    n = x.shape[0]
    grid = n // _ROWB
    inv_n = 1.0 / n_nodes

    def body(s2_ref, x_ref, d2_ref, lw_ref, lb_ref, g_ref, bt_ref,
             out_ref, h_scr, sacc, qacc):
        p = pl.program_id(0)
        i = pl.program_id(1)

        @pl.when(p == 0)
        def _():
            xt = jnp.dot(x_ref[...], lw_ref[...],
                         preferred_element_type=jnp.float32) + lb_ref[...]
            sums = jnp.concatenate([s2_ref[0], s2_ref[1]], axis=-1) + xt
            cnt = jnp.concatenate([d2_ref[0], d2_ref[1]], axis=-1) + 1.0
            h = sums / cnt + x_ref[...]
            h_scr[pl.ds(i * _ROWB, _ROWB), :] = h

            @pl.when(i == 0)
            def _():
                sacc[...] = jnp.zeros_like(sacc)
                qacc[...] = jnp.zeros_like(qacc)

            sacc[...] += jnp.sum(h, axis=0, keepdims=True)
            qacc[...] += jnp.sum(h * h, axis=0, keepdims=True)

        @pl.when(p == 1)
        def _():
            mu = sacc[...] * inv_n
            var = qacc[...] * inv_n - mu * mu
            h = h_scr[pl.ds(i * _ROWB, _ROWB), :]
            hn = (h - mu) * lax.rsqrt(var + 1e-5)
            out_ref[...] = _gelu(hn * g_ref[...] + bt_ref[...])

    first = lambda p, i: jnp.where(p == 0, i, 0)
    return pl.pallas_call(
        body,
        grid=(2, grid),
        in_specs=[
            pl.BlockSpec((2, _ROWB, 32), lambda p, i: (0, first(p, i), 0)),
            pl.BlockSpec((_ROWB, 64), lambda p, i: (first(p, i), 0)),
            pl.BlockSpec((2, _ROWB, 32), lambda p, i: (0, first(p, i), 0)),
            pl.BlockSpec((64, 64), lambda p, i: (0, 0)),
            pl.BlockSpec((1, 64), lambda p, i: (0, 0)),
            pl.BlockSpec((1, 64), lambda p, i: (0, 0)),
            pl.BlockSpec((1, 64), lambda p, i: (0, 0)),
        ],
        out_specs=pl.BlockSpec((_ROWB, 64), lambda p, i: (i, 0)),
        out_shape=jax.ShapeDtypeStruct((n, 64), jnp.float32),
        scratch_shapes=[
            pltpu.VMEM((n, 64), jnp.float32),
            pltpu.VMEM((1, 64), jnp.float32),
            pltpu.VMEM((1, 64), jnp.float32),
        ],
    )(sums2, x, deg2, lw, lb.reshape(1, 64), gamma, beta)


def _tc_gin(x1, agg2, w1, b1, w2, b2, lw=None, lb=None):
    n = x1.shape[0]
    grid = n // _ROWB
    with_lin = lw is not None

    def body(x_ref, a2_ref, w1_ref, b1_ref, w2_ref, b2_ref, *rest):
        z = x_ref[...] + jnp.concatenate([a2_ref[0], a2_ref[1]], axis=-1)
        z1 = _gelu(jnp.dot(z, w1_ref[...],
                           preferred_element_type=jnp.float32) + b1_ref[...])
        x2 = jnp.dot(z1, w2_ref[...],
                     preferred_element_type=jnp.float32) + b2_ref[...]
        if with_lin:
            lw_ref, lb_ref, out_ref, tab_ref = rest
            t = jnp.dot(x2, lw_ref[...],
                        preferred_element_type=jnp.float32) + lb_ref[...]
            tab_ref[0] = t
            tab_ref[1] = -t
        else:
            (out_ref,) = rest
        out_ref[...] = x2

    in_specs = [
        pl.BlockSpec((_ROWB, 64), lambda i: (i, 0)),
        pl.BlockSpec((2, _ROWB, 32), lambda i: (0, i, 0)),
        pl.BlockSpec((64, 64), lambda i: (0, 0)),
        pl.BlockSpec((1, 64), lambda i: (0, 0)),
        pl.BlockSpec((64, 64), lambda i: (0, 0)),
        pl.BlockSpec((1, 64), lambda i: (0, 0)),
    ]
    args = [x1, agg2, w1, b1.reshape(1, 64), w2, b2.reshape(1, 64)]
    out_specs = [pl.BlockSpec((_ROWB, 64), lambda i: (i, 0))]
    out_shape = [jax.ShapeDtypeStruct((n, 64), jnp.float32)]
    if with_lin:
        in_specs += [pl.BlockSpec((64, 64), lambda i: (0, 0)),
                     pl.BlockSpec((1, 64), lambda i: (0, 0))]
        args += [lw, lb.reshape(1, 64)]
        out_specs += [pl.BlockSpec((2, _ROWB, 64), lambda i: (0, i, 0))]
        out_shape += [jax.ShapeDtypeStruct((2, n, 64), jnp.float32)]

    return pl.pallas_call(
        body,
        grid=(grid,),
        in_specs=in_specs,
        out_specs=out_specs,
        out_shape=out_shape,
    )(*args)


def _tc_fuse_readout(outs, fw, fb, ro):
    n = outs[0].shape[0]
    grid = n // _ROWB

    def ln(x, w, b):
        m = jnp.mean(x, axis=-1, keepdims=True)
        v = jnp.mean((x - m) * (x - m), axis=-1, keepdims=True)
        return (x - m) * lax.rsqrt(v + 1e-5) * w + b

    def body(o0, o1, o2, o3, o4, o5, fw_ref, fb_ref,
             w1_ref, b1_ref, l1w, l1b, w2_ref, b2_ref, l2w, l2b,
             w3_ref, b3_ref, xf_ref, p_ref):
        os_ = (o0, o1, o2, o3, o4, o5)
        xf = fb_ref[...] + jnp.zeros((o0.shape[0], 64), jnp.float32)
        for j in range(6):
            xf = xf + jnp.dot(os_[j][...], fw_ref[j],
                              preferred_element_type=jnp.float32)
        xf_ref[...] = xf
        h1 = jnp.maximum(
            ln(jnp.dot(xf, w1_ref[...], preferred_element_type=jnp.float32)
               + b1_ref[...], l1w[...], l1b[...]), 0.0)
        h2 = jnp.maximum(
            ln(jnp.dot(h1, w2_ref[...], preferred_element_type=jnp.float32)
               + b2_ref[...], l2w[...], l2b[...]), 0.0)
        logit = jnp.dot(h2, w3_ref[...],
                        preferred_element_type=jnp.float32) + b3_ref[...]
        p_ref[...] = 1.0 / (1.0 + jnp.exp(-logit))

    full = lambda shape: pl.BlockSpec(shape, lambda i: tuple(0 for _ in shape))
    rowspec = pl.BlockSpec((_ROWB, 64), lambda i: (i, 0))
    return pl.pallas_call(
        body,
        grid=(grid,),
        in_specs=[rowspec] * 6 + [
            full((6, 64, 64)), full((1, 64)),
            full((64, 128)), full((1, 128)), full((1, 128)), full((1, 128)),
            full((128, 128)), full((1, 128)), full((1, 128)), full((1, 128)),
            full((128, 1)), full((1, 1)),
        ],
        out_specs=[
            pl.BlockSpec((_ROWB, 64), lambda i: (i, 0)),
            pl.BlockSpec((_ROWB, 1), lambda i: (i, 0)),
        ],
        out_shape=[
            jax.ShapeDtypeStruct((n, 64), jnp.float32),
            jax.ShapeDtypeStruct((n, 1), jnp.float32),
        ],
    )(*outs, fw, fb.reshape(1, 64),
      ro['W1'], ro['b1'].reshape(1, 128), ro['ln1_w'].reshape(1, 128),
      ro['ln1_b'].reshape(1, 128),
      ro['W2'], ro['b2'].reshape(1, 128), ro['ln2_w'].reshape(1, 128),
      ro['ln2_b'].reshape(1, 128),
      ro['W3'], ro['b3'].reshape(1, 1))


def kernel(init_emb, edge_index_s, rate_b, params):
    n, d = init_emb.shape
    e = edge_index_s.shape[0]
    assert d == 64 and n % _ROWB == 0 and n % _NS == 0

    src = edge_index_s[:, 0].astype(jnp.int32)
    dst = edge_index_s[:, 1].astype(jnp.int32)
    sign = edge_index_s[:, 2].astype(jnp.int32)

    grain = _NS * _CHUNK * _K
    e_pad = ((e + grain - 1) // grain) * grain
    pad = e_pad - e
    src_p = jnp.concatenate([src, jnp.zeros((pad,), jnp.int32)])
    dst_p = jnp.concatenate([dst, jnp.full((pad,), n, jnp.int32)])
    sign_p = jnp.concatenate([sign, jnp.ones((pad,), jnp.int32)])
    neg = (sign_p < 0).astype(jnp.int32)

    n_chunk_rows = e_pad // _CHUNK
    cc = jnp.arange(_NC, dtype=jnp.int32).reshape(_NC, 1)
    src_sage = (((src_p + n * neg) * 2)[None, :] + cc).reshape(
        _NC, n_chunk_rows, _CHUNK)
    src_gin = ((src_p * 2)[None, :] + cc).reshape(_NC, n_chunk_rows, _CHUNK)
    dst_t = dst_p.reshape(n_chunk_rows, _CHUNK)
    dst2 = jnp.broadcast_to(dst_t[None], (_NC, n_chunk_rows, _CHUNK))
    sd_sage = jnp.stack([src_sage, dst2], axis=2)
    sd_gin = jnp.stack([src_gin, dst2], axis=2)

    deg2 = _sc_degree(dst_t, n)

    rb = rate_b.reshape(1, 1)
    x = init_emb
    outs = []
    taba = _tc_lin_stack(x, params['sage0']['lin_W'], params['sage0']['lin_b'])
    for i in range(3):
        sp = params['sage%d' % i]
        sums2 = _sc_segment_sum(taba.reshape(4 * n, 32), sd_sage, n)
        gamma = (sp['norm_w'][None, :] + rb @ sp['rs_W'] + sp['rs_b'][None, :])
        beta = (sp['norm_b'][None, :] + rb @ sp['rb_W'] + sp['rb_b'][None, :])
        x1 = _tc_sage_dense(sums2, x, deg2, sp['lin_W'], sp['lin_b'],
                            gamma, beta, n)
        outs.append(x1)

        gp = params['gin%d' % i]
        agg2 = _sc_segment_sum(x1.reshape(n * 2, 32), sd_gin, n)
        if i < 2:
            nsp = params['sage%d' % (i + 1)]
            x2, taba = _tc_gin(x1, agg2, gp['W1'], gp['b1'], gp['W2'],
                               gp['b2'], nsp['lin_W'], nsp['lin_b'])
        else:
            (x2,) = _tc_gin(x1, agg2, gp['W1'], gp['b1'], gp['W2'], gp['b2'])
        outs.append(x2)
        x = x2

    fw = params['fuse_W'].reshape(6, 64, 64)
    x_final, prob = _tc_fuse_readout(outs, fw, params['fuse_b'], params['ro'])
    return x_final, prob
